# baseline (speedup 1.0000x reference)
import functools
import jax
import jax.numpy as jnp
from jax import lax
from jax.experimental import pallas as pl
from jax.experimental.pallas import tpu as pltpu
from jax.experimental.pallas import tpu_sc as plsc

D_MODEL = 768
VOCAB = 100000
TOKENS = 4 * 2048
NUM_WORKERS = 32
TPW = TOKENS // NUM_WORKERS
G = 32
NG = TPW // G
DCH = D_MODEL // 16


def _body(x_hbm, w_hbm, out_hbm, x_vm, idx_vm, dat_vm, sem):
    cid = lax.axis_index("c")
    sid = lax.axis_index("s")
    wid = sid * 2 + cid
    base = wid * TPW
    pltpu.sync_copy(x_hbm.at[pl.ds(base, TPW)], x_vm)

    iota16 = lax.iota(jnp.int32, 16)

    def group(g, _):
        def tokvec(v, _):
            tv = x_vm[pl.ds(g * G + v * 16, 16)]
            for l in range(16):
                xv = tv[l]
                t = v * 16 + l
                for j in range(DCH):
                    idx_vm[pl.ds((t * DCH + j) * 16, 16)] = (
                        iota16 * VOCAB + (j * 16 * VOCAB + xv)
                    )
            return 0

        lax.fori_loop(0, G // 16, tokvec, 0)
        pltpu.async_copy(w_hbm.at[idx_vm], dat_vm, sem).wait()
        pltpu.sync_copy(
            dat_vm, out_hbm.at[pl.ds((base + g * G) * D_MODEL, G * D_MODEL)]
        )
        return 0

    lax.fori_loop(0, NG, group, 0)


@jax.jit
def _embed(xf, wf):
    mesh = plsc.VectorSubcoreMesh(core_axis_name="c", subcore_axis_name="s")
    f = functools.partial(
        pl.kernel,
        mesh=mesh,
        out_type=jax.ShapeDtypeStruct((TOKENS * D_MODEL,), jnp.float32),
        scratch_types=[
            pltpu.VMEM((TPW,), jnp.int32),
            pltpu.VMEM((G * D_MODEL,), jnp.int32),
            pltpu.VMEM((G * D_MODEL,), jnp.float32),
            pltpu.SemaphoreType.DMA,
        ],
    )(_body)
    return f(xf, wf)


def kernel(x, W_E):
    xf = x.reshape(TOKENS).astype(jnp.int32)
    wf = W_E.reshape(D_MODEL * VOCAB)
    out = _embed(xf, wf)
    return out.reshape(4, 2048, D_MODEL)


# speedup vs baseline: 24.3084x; 24.3084x over previous
import functools
import jax
import jax.numpy as jnp
from jax import lax
from jax.experimental import pallas as pl
from jax.experimental.pallas import tpu as pltpu
from jax.experimental.pallas import tpu_sc as plsc

D_MODEL = 768
VOCAB = 100000
TOKENS = 4 * 2048
NUM_WORKERS = 32
TPW = TOKENS // NUM_WORKERS
CH = 64
NCH = TPW // CH


def _body(x_hbm, wt_hbm, out_hbm, x_vm, rows0, rows1, sem0, sem1):
    cid = lax.axis_index("c")
    sid = lax.axis_index("s")
    wid = sid * 2 + cid
    base = wid * TPW
    pltpu.sync_copy(x_hbm.at[pl.ds(wid * NCH, NCH)], x_vm)

    bufs = (rows0, rows1)
    sems = (sem0, sem1)
    copies = []
    for c in range(NCH):
        copies.append(
            pltpu.async_copy(wt_hbm.at[x_vm.at[c]], bufs[c % 2], sems[c % 2])
        )
        if c >= 1:
            copies[c - 1].wait()
            pltpu.sync_copy(
                bufs[(c - 1) % 2], out_hbm.at[pl.ds(base + (c - 1) * CH, CH)]
            )
    copies[NCH - 1].wait()
    pltpu.sync_copy(
        bufs[(NCH - 1) % 2], out_hbm.at[pl.ds(base + (NCH - 1) * CH, CH)]
    )


@jax.jit
def _embed(xf, wt):
    mesh = plsc.VectorSubcoreMesh(core_axis_name="c", subcore_axis_name="s")
    f = functools.partial(
        pl.kernel,
        mesh=mesh,
        out_type=jax.ShapeDtypeStruct((TOKENS, D_MODEL), jnp.float32),
        scratch_types=[
            pltpu.VMEM((NCH, CH), jnp.int32),
            pltpu.VMEM((CH, D_MODEL), jnp.float32),
            pltpu.VMEM((CH, D_MODEL), jnp.float32),
            pltpu.SemaphoreType.DMA,
            pltpu.SemaphoreType.DMA,
        ],
    )(_body)
    return f(xf, wt)


def kernel(x, W_E):
    xf = x.reshape(TOKENS // CH, CH).astype(jnp.int32)
    wt = W_E.T
    out = _embed(xf, wt)
    return out.reshape(4, 2048, D_MODEL)


# speedup vs baseline: 24.8833x; 1.0237x over previous
import functools
import jax
import jax.numpy as jnp
from jax import lax
from jax.experimental import pallas as pl
from jax.experimental.pallas import tpu as pltpu
from jax.experimental.pallas import tpu_sc as plsc

D_MODEL = 768
VOCAB = 100000
TOKENS = 4 * 2048
NUM_WORKERS = 32
TPW = TOKENS // NUM_WORKERS
CH = 32
NCH = TPW // CH
NBUF = 4


def _body(x_hbm, wt_hbm, out_hbm, x_vm, b0, b1, b2, b3, g0, g1, g2, g3,
          w0, w1, w2, w3):
    cid = lax.axis_index("c")
    sid = lax.axis_index("s")
    wid = sid * 2 + cid
    base = wid * TPW
    pltpu.sync_copy(x_hbm.at[pl.ds(wid * NCH, NCH)], x_vm)

    bufs = (b0, b1, b2, b3)
    gsems = (g0, g1, g2, g3)
    wsems = (w0, w1, w2, w3)
    gathers = [None] * NCH
    writes = [None] * NCH
    for c in range(NBUF):
        gathers[c] = pltpu.async_copy(
            wt_hbm.at[x_vm.at[c]], bufs[c], gsems[c]
        )
    for c in range(NCH):
        k = c % NBUF
        gathers[c].wait()
        writes[c] = pltpu.async_copy(
            bufs[k], out_hbm.at[pl.ds(base + c * CH, CH)], wsems[k]
        )
        if c + NBUF < NCH:
            writes[c].wait()
            gathers[c + NBUF] = pltpu.async_copy(
                wt_hbm.at[x_vm.at[c + NBUF]], bufs[k], gsems[k]
            )
    for c in range(NCH - NBUF, NCH):
        if writes[c] is not None:
            writes[c].wait()


@jax.jit
def _embed(xf, wt):
    mesh = plsc.VectorSubcoreMesh(core_axis_name="c", subcore_axis_name="s")
    f = functools.partial(
        pl.kernel,
        mesh=mesh,
        out_type=jax.ShapeDtypeStruct((TOKENS, D_MODEL), jnp.float32),
        scratch_types=(
            [pltpu.VMEM((NCH, CH), jnp.int32)]
            + [pltpu.VMEM((CH, D_MODEL), jnp.float32)] * NBUF
            + [pltpu.SemaphoreType.DMA] * (2 * NBUF)
        ),
    )(_body)
    return f(xf, wt)


def kernel(x, W_E):
    xf = x.reshape(TOKENS // CH, CH).astype(jnp.int32)
    wt = W_E.T
    out = _embed(xf, wt)
    return out.reshape(4, 2048, D_MODEL)


# speedup vs baseline: 24.9003x; 1.0007x over previous
import functools
import jax
import jax.numpy as jnp
from jax import lax
from jax.experimental import pallas as pl
from jax.experimental.pallas import tpu as pltpu
from jax.experimental.pallas import tpu_sc as plsc

D_MODEL = 768
VOCAB = 100000
TOKENS = 4 * 2048
NUM_WORKERS = 32
TPW = TOKENS // NUM_WORKERS
CH = 16
NCH = TPW // CH
NBUF = 8


def _body(x_hbm, wt_hbm, out_hbm, x_vm, *rest):
    cid = lax.axis_index("c")
    sid = lax.axis_index("s")
    wid = sid * 2 + cid
    base = wid * TPW
    pltpu.sync_copy(x_hbm.at[pl.ds(wid * NCH, NCH)], x_vm)

    bufs = rest[:NBUF]
    gsems = rest[NBUF:2 * NBUF]
    wsems = rest[2 * NBUF:]
    gathers = [None] * NCH
    writes = [None] * NCH
    for c in range(NBUF):
        gathers[c] = pltpu.async_copy(
            wt_hbm.at[x_vm.at[c]], bufs[c], gsems[c]
        )
    for c in range(NCH):
        k = c % NBUF
        gathers[c].wait()
        writes[c] = pltpu.async_copy(
            bufs[k], out_hbm.at[pl.ds(base + c * CH, CH)], wsems[k]
        )
        if c + NBUF < NCH:
            writes[c].wait()
            gathers[c + NBUF] = pltpu.async_copy(
                wt_hbm.at[x_vm.at[c + NBUF]], bufs[k], gsems[k]
            )
    for c in range(NCH - NBUF, NCH):
        if writes[c] is not None:
            writes[c].wait()


@jax.jit
def _embed(xf, wt):
    mesh = plsc.VectorSubcoreMesh(core_axis_name="c", subcore_axis_name="s")
    f = functools.partial(
        pl.kernel,
        mesh=mesh,
        out_type=jax.ShapeDtypeStruct((TOKENS, D_MODEL), jnp.float32),
        scratch_types=(
            [pltpu.VMEM((NCH, CH), jnp.int32)]
            + [pltpu.VMEM((CH, D_MODEL), jnp.float32)] * NBUF
            + [pltpu.SemaphoreType.DMA] * (2 * NBUF)
        ),
    )(_body)
    return f(xf, wt)


def kernel(x, W_E):
    xf = x.reshape(TOKENS // CH, CH).astype(jnp.int32)
    wt = W_E.T
    out = _embed(xf, wt)
    return out.reshape(4, 2048, D_MODEL)
